# baseline (speedup 1.0000x reference)
import functools
import jax
import jax.numpy as jnp
from jax import lax
from jax.experimental import pallas as pl
from jax.experimental.pallas import tpu as pltpu, tpu_sc as plsc

W = 20
B = 4096
DIM = 64
L = 16

_NC, _NS = 2, 16
NW = _NC * _NS
BPW = B // NW
GROUPS = BPW // L
DCH = DIM // L

N_ROWS = 1000001
PACK_BLK = 1024
N_BLKS = -(-N_ROWS // PACK_BLK)
N_WIDE = N_BLKS * PACK_BLK


def _widen_kernel(tab_ref, out_ref):
    x = tab_ref[...]
    out_ref[...] = jnp.concatenate([x, x], axis=1)


def _widen(table):
    return pl.pallas_call(
        _widen_kernel,
        grid=(N_BLKS,),
        in_specs=[pl.BlockSpec((PACK_BLK, DIM), lambda i: (i, 0))],
        out_specs=pl.BlockSpec((PACK_BLK, 2 * DIM), lambda i: (i, 0)),
        out_shape=jax.ShapeDtypeStruct((N_WIDE, 2 * DIM), jnp.float32),
    )(table)


def _splat_lane(v, j):
    idx = jnp.full((L, 1), j, dtype=jnp.int32)
    dnums = lax.GatherDimensionNumbers(
        offset_dims=(), collapsed_slice_dims=(0,), start_index_map=(0,))
    return lax.gather(v, idx, dnums, slice_sizes=(1,),
                      mode=lax.GatherScatterMode.PROMISE_IN_BOUNDS)


def _build(interpret=False):
    mesh = plsc.VectorSubcoreMesh(
        core_axis_name="c", subcore_axis_name="s",
        num_cores=_NC, num_subcores=_NS)

    @functools.partial(
        pl.kernel,
        out_type=jax.ShapeDtypeStruct((B, DIM), jnp.float32),
        mesh=mesh,
        scratch_types=[
            pltpu.VMEM((W, BPW), jnp.int32),
            pltpu.VMEM((W, BPW), jnp.float32),
            pltpu.VMEM((W, BPW), jnp.float32),
            pltpu.VMEM((W * BPW,), jnp.float32),
            pltpu.VMEM((BPW, 2 * DIM), jnp.float32),
            pltpu.VMEM((BPW, 2 * DIM), jnp.float32),
            pltpu.VMEM((BPW, DIM), jnp.float32),
            pltpu.SemaphoreType.DMA,
            pltpu.SemaphoreType.DMA,
        ],
        compiler_params=pltpu.CompilerParams(use_tc_tiling_on_sc=True),
        interpret=interpret,
    )
    def sc_kernel(table_hbm, idx_hbm, rew_hbm, live_hbm, out_hbm,
                  idx_v, rew_v, live_v, wts_v, rows0, rows1, acc_v,
                  sem0, sem1):
        wid = lax.axis_index("s") * _NC + lax.axis_index("c")
        base = wid * BPW

        pltpu.sync_copy(idx_hbm.at[:, pl.ds(base, BPW)], idx_v)
        pltpu.sync_copy(rew_hbm.at[:, pl.ds(base, BPW)], rew_v)
        pltpu.sync_copy(live_hbm.at[:, pl.ds(base, BPW)], live_v)

        rows = (rows0, rows1)
        sems = (sem0, sem1)
        cp0 = pltpu.make_async_copy(table_hbm.at[idx_v.at[0]], rows0, sem0)
        cp0.start()

        for c in range(GROUPS):
            sl = pl.ds(c * L, L)
            cnt = live_v[0, sl]
            for w in range(1, W):
                cnt = cnt + live_v[w, sl]
            rcp = 1.0 / cnt
            for w in range(W):
                wts_v[pl.ds(w * BPW + c * L, L)] = rew_v[w, sl] * live_v[w, sl] * rcp

        pending = cp0
        for w in range(W):
            pending.wait()
            if w + 1 < W:
                pending = pltpu.make_async_copy(
                    table_hbm.at[idx_v.at[w + 1]], rows[(w + 1) % 2], sems[(w + 1) % 2])
                pending.start()
            rbuf = rows[w % 2]

            def group_body(g, _, w=w, rbuf=rbuf):
                w16 = wts_v[pl.ds(w * BPW + g * L, L)]
                for j in range(L):
                    wv = _splat_lane(w16, j)
                    r = g * L + j
                    for d in range(DCH):
                        sl = pl.ds(d * L, L)
                        prod = rbuf[r, sl] * wv
                        if w == 0:
                            acc_v[r, sl] = prod
                        else:
                            plsc.addupdate(acc_v.at[r, sl], prod)
                return 0

            lax.fori_loop(0, GROUPS, group_body, 0)

        pltpu.sync_copy(acc_v, out_hbm.at[pl.ds(base, BPW)])

    return sc_kernel


_sc_kernel = None


def kernel(item_table, indices, rew, live_mat):
    global _sc_kernel
    if _sc_kernel is None:
        _sc_kernel = _build()
    wide = _widen(item_table)
    idx2 = indices.reshape(W, B)
    rew2 = rew.reshape(W, B)
    live_f = live_mat.astype(jnp.float32)
    return _sc_kernel(wide, idx2, rew2, live_f)


# speedup vs baseline: 1.5561x; 1.5561x over previous
import functools
import jax
import jax.numpy as jnp
from jax import lax
from jax.experimental import pallas as pl
from jax.experimental.pallas import tpu as pltpu, tpu_sc as plsc

W = 20
B = 4096
DIM = 64
L = 16

_NC, _NS = 2, 16
NW = _NC * _NS
BPW = B // NW
GROUPS = BPW // L
DCH = DIM // L

N_ROWS = 1000001
PACK_BLK = 1024
N_BLKS = -(-N_ROWS // PACK_BLK)
N_WIDE = N_BLKS * PACK_BLK


def _widen_kernel(tab_ref, out_ref):
    x = tab_ref[...]
    out_ref[...] = jnp.concatenate([x, x], axis=1)


def _widen(table):
    return pl.pallas_call(
        _widen_kernel,
        grid=(N_BLKS,),
        in_specs=[pl.BlockSpec((PACK_BLK, DIM), lambda i: (i, 0))],
        out_specs=pl.BlockSpec((PACK_BLK, 2 * DIM), lambda i: (i, 0)),
        out_shape=jax.ShapeDtypeStruct((N_WIDE, 2 * DIM), jnp.float32),
    )(table)


def _splat_lane(v, j):
    idx = jnp.full((L, 1), j, dtype=jnp.int32)
    dnums = lax.GatherDimensionNumbers(
        offset_dims=(), collapsed_slice_dims=(0,), start_index_map=(0,))
    return lax.gather(v, idx, dnums, slice_sizes=(1,),
                      mode=lax.GatherScatterMode.PROMISE_IN_BOUNDS)


def _build(interpret=False):
    mesh = plsc.VectorSubcoreMesh(
        core_axis_name="c", subcore_axis_name="s",
        num_cores=_NC, num_subcores=_NS)

    @functools.partial(
        pl.kernel,
        out_type=jax.ShapeDtypeStruct((B, DIM), jnp.float32),
        mesh=mesh,
        scratch_types=[
            pltpu.VMEM((W, BPW), jnp.int32),
            pltpu.VMEM((W, BPW), jnp.float32),
            pltpu.VMEM((W, BPW), jnp.float32),
            pltpu.VMEM((W * BPW,), jnp.float32),
            pltpu.VMEM((BPW, 2 * DIM), jnp.float32),
            pltpu.VMEM((BPW, 2 * DIM), jnp.float32),
            pltpu.VMEM((BPW, DIM), jnp.float32),
            pltpu.SemaphoreType.DMA,
            pltpu.SemaphoreType.DMA,
        ],
        compiler_params=pltpu.CompilerParams(use_tc_tiling_on_sc=True),
        interpret=interpret,
    )
    def sc_kernel(table_hbm, idx_hbm, rew_hbm, live_hbm, out_hbm,
                  idx_v, rew_v, live_v, wts_v, rows0, rows1, acc_v,
                  sem0, sem1):
        wid = lax.axis_index("s") * _NC + lax.axis_index("c")
        base = wid * BPW

        pltpu.sync_copy(idx_hbm.at[:, pl.ds(base, BPW)], idx_v)
        pltpu.sync_copy(rew_hbm.at[:, pl.ds(base, BPW)], rew_v)
        pltpu.sync_copy(live_hbm.at[:, pl.ds(base, BPW)], live_v)

        rows = (rows0, rows1)
        sems = (sem0, sem1)
        cp0 = pltpu.make_async_copy(table_hbm.at[idx_v.at[0]], rows0, sem0)
        cp0.start()

        for c in range(GROUPS):
            sl = pl.ds(c * L, L)
            cnt = live_v[0, sl]
            for w in range(1, W):
                cnt = cnt + live_v[w, sl]
            rcp = 1.0 / cnt
            for w in range(W):
                wts_v[pl.ds(w * BPW + c * L, L)] = rew_v[w, sl] * live_v[w, sl] * rcp

        pending = cp0
        for w in range(W):
            pending.wait()
            if w + 1 < W:
                pending = pltpu.make_async_copy(
                    table_hbm.at[idx_v.at[w + 1]], rows[(w + 1) % 2], sems[(w + 1) % 2])
                pending.start()
            rbuf = rows[w % 2]

            def group_body(g, _, w=w, rbuf=rbuf):
                w16 = wts_v[pl.ds(w * BPW + g * L, L)]
                for j in range(L):
                    wv = _splat_lane(w16, j)
                    r = g * L + j
                    for d in range(DCH):
                        sl = pl.ds(d * L, L)
                        prod = rbuf[r, sl] * wv
                        if w == 0:
                            acc_v[r, sl] = prod
                        else:
                            plsc.addupdate(acc_v.at[r, sl], prod)
                return 0

            lax.fori_loop(0, GROUPS, group_body, 0)

        pltpu.sync_copy(acc_v, out_hbm.at[pl.ds(base, BPW)])

    return sc_kernel


_sc_kernel = None


def kernel(item_table, indices, rew, live_mat):
    global _sc_kernel
    if _sc_kernel is None:
        _sc_kernel = _build()
    wide = jnp.concatenate([item_table, item_table], axis=1)
    idx2 = indices.reshape(W, B)
    rew2 = rew.reshape(W, B)
    live_f = live_mat.astype(jnp.float32)
    return _sc_kernel(wide, idx2, rew2, live_f)


# speedup vs baseline: 1.7883x; 1.1492x over previous
import functools
import jax
import jax.numpy as jnp
from jax import lax
from jax.experimental import pallas as pl
from jax.experimental.pallas import tpu as pltpu, tpu_sc as plsc

W = 20
B = 4096
DIM = 64
L = 16

_NC, _NS = 2, 16
NW = _NC * _NS
BPW = B // NW
GROUPS = BPW // L
DCH = DIM // L


def _splat_lane(v, j):
    idx = jnp.full((L, 1), j, dtype=jnp.int32)
    dnums = lax.GatherDimensionNumbers(
        offset_dims=(), collapsed_slice_dims=(0,), start_index_map=(0,))
    return lax.gather(v, idx, dnums, slice_sizes=(1,),
                      mode=lax.GatherScatterMode.PROMISE_IN_BOUNDS)


def _build(interpret=False):
    mesh = plsc.VectorSubcoreMesh(
        core_axis_name="c", subcore_axis_name="s",
        num_cores=_NC, num_subcores=_NS)

    @functools.partial(
        pl.kernel,
        out_type=jax.ShapeDtypeStruct((B, DIM), jnp.float32),
        mesh=mesh,
        scratch_types=[
            pltpu.VMEM((W, BPW), jnp.int32),
            pltpu.VMEM((W, BPW), jnp.float32),
            pltpu.VMEM((W, BPW), jnp.float32),
            pltpu.VMEM((W * BPW,), jnp.float32),
            pltpu.VMEM((BPW, DIM), jnp.float32),
            pltpu.VMEM((BPW, DIM), jnp.float32),
            pltpu.VMEM((BPW, DIM), jnp.float32),
            pltpu.SemaphoreType.DMA,
            pltpu.SemaphoreType.DMA,
        ],
        compiler_params=pltpu.CompilerParams(use_tc_tiling_on_sc=False),
        interpret=interpret,
    )
    def sc_kernel(table_hbm, idx_hbm, rew_hbm, live_hbm, out_hbm,
                  idx_v, rew_v, live_v, wts_v, rows0, rows1, acc_v,
                  sem0, sem1):
        wid = lax.axis_index("s") * _NC + lax.axis_index("c")
        base = wid * BPW

        pltpu.sync_copy(idx_hbm.at[:, pl.ds(base, BPW)], idx_v)
        pltpu.sync_copy(rew_hbm.at[:, pl.ds(base, BPW)], rew_v)
        pltpu.sync_copy(live_hbm.at[:, pl.ds(base, BPW)], live_v)

        rows = (rows0, rows1)
        sems = (sem0, sem1)
        cp0 = pltpu.make_async_copy(table_hbm.at[idx_v.at[0]], rows0, sem0)
        cp0.start()

        for c in range(GROUPS):
            sl = pl.ds(c * L, L)
            cnt = live_v[0, sl]
            for w in range(1, W):
                cnt = cnt + live_v[w, sl]
            rcp = 1.0 / cnt
            for w in range(W):
                wts_v[pl.ds(w * BPW + c * L, L)] = rew_v[w, sl] * live_v[w, sl] * rcp

        pending = cp0
        for w in range(W):
            pending.wait()
            if w + 1 < W:
                pending = pltpu.make_async_copy(
                    table_hbm.at[idx_v.at[w + 1]], rows[(w + 1) % 2], sems[(w + 1) % 2])
                pending.start()
            rbuf = rows[w % 2]

            def group_body(g, _, w=w, rbuf=rbuf):
                w16 = wts_v[pl.ds(w * BPW + g * L, L)]
                for j in range(L):
                    wv = _splat_lane(w16, j)
                    r = g * L + j
                    for d in range(DCH):
                        sl = pl.ds(d * L, L)
                        prod = rbuf[r, sl] * wv
                        if w == 0:
                            acc_v[r, sl] = prod
                        else:
                            plsc.addupdate(acc_v.at[r, sl], prod)
                return 0

            lax.fori_loop(0, GROUPS, group_body, 0)

        pltpu.sync_copy(acc_v, out_hbm.at[pl.ds(base, BPW)])

    return sc_kernel


_sc_kernel = None


def kernel(item_table, indices, rew, live_mat):
    global _sc_kernel
    if _sc_kernel is None:
        _sc_kernel = _build()
    idx2 = indices.reshape(W, B)
    rew2 = rew.reshape(W, B)
    live_f = live_mat.astype(jnp.float32)
    return _sc_kernel(item_table, idx2, rew2, live_f)
